# baseline (speedup 1.0000x reference)
import functools

import jax
import jax.numpy as jnp
from jax import lax
from jax.experimental import pallas as pl
from jax.experimental.pallas import tpu as pltpu
from jax.experimental.pallas import tpu_sc as plsc

D = 64
B = 16384


def _sc_gather_pairs(table2, idx):
    info = plsc.get_sparse_core_info()
    nc, ns = info.num_cores, info.num_subcores
    nw = nc * ns
    b_per_w = B // nw

    mesh = plsc.VectorSubcoreMesh(core_axis_name="c", subcore_axis_name="s")

    @functools.partial(
        pl.kernel,
        mesh=mesh,
        out_type=jax.ShapeDtypeStruct((B, D), jnp.float32),
        scratch_types=[
            pltpu.VMEM((b_per_w,), jnp.int32),
            pltpu.VMEM((b_per_w // 2, 2 * D), jnp.float32),
            pltpu.VMEM((b_per_w // 2, 2 * D), jnp.float32),
            pltpu.VMEM((b_per_w // 2, D), jnp.float32),
            pltpu.SemaphoreType.DMA,
            pltpu.SemaphoreType.DMA,
        ],
    )
    def k(table_hbm, idx_hbm, out_hbm, idx_v, pairs0, pairs1, rows_v,
          sem0, sem1):
        ch = b_per_w // 2
        wid = lax.axis_index("s") * nc + lax.axis_index("c")
        base = wid * b_per_w
        pltpu.sync_copy(idx_hbm.at[pl.ds(base, b_per_w)], idx_v)

        def mk_fire(pairs, sem, goff):
            def fire(g, _):
                vec = idx_v[pl.ds((goff + g) * 16, 16)]
                for lane in range(16):
                    u = lax.shift_right_logical(vec[lane], 1)
                    pltpu.make_async_copy(
                        table_hbm.at[u], pairs.at[g * 16 + lane], sem
                    ).start()
                return ()
            return fire

        ngrp = ch // 16
        lax.fori_loop(0, ngrp, mk_fire(pairs0, sem0, 0), ())
        lax.fori_loop(0, ngrp, mk_fire(pairs1, sem1, ngrp), ())

        for c, pairs, sem in ((0, pairs0, sem0), (1, pairs1, sem1)):
            pltpu.make_async_copy(
                table_hbm.at[pl.ds(0, ch)], pairs, sem
            ).wait()

            def extract(g, _, pairs=pairs, c=c):
                vec = idx_v[pl.ds((c * ngrp + g) * 16, 16)]
                for lane in range(16):
                    j = g * 16 + lane
                    off = lax.bitwise_and(vec[lane], 1) * D
                    for q in range(D // 16):
                        rows_v[j, pl.ds(q * 16, 16)] = (
                            pairs[j, pl.ds(off + q * 16, 16)]
                        )
                return ()

            lax.fori_loop(0, ngrp, extract, ())
            pltpu.sync_copy(rows_v, out_hbm.at[pl.ds(base + c * ch, ch)])

    return k(table2, idx)


def _tc_project(emb, W, b2d):
    blk = 2048

    def body(emb_ref, w_ref, b_ref, out_ref):
        acc = lax.dot_general(
            emb_ref[...], w_ref[...],
            (((1,), (1,)), ((), ())),
            preferred_element_type=jnp.float32,
        )
        out_ref[...] = acc + b_ref[...]

    return pl.pallas_call(
        body,
        grid=(B // blk,),
        in_specs=[
            pl.BlockSpec((blk, D), lambda i: (i, 0)),
            pl.BlockSpec((D, D), lambda i: (0, 0)),
            pl.BlockSpec((1, D), lambda i: (0, 0)),
        ],
        out_specs=pl.BlockSpec((blk, D), lambda i: (i, 0)),
        out_shape=jax.ShapeDtypeStruct((B, D), jnp.float32),
    )(emb, W, b2d)


def kernel(l, table, W, b):
    idx = l.astype(jnp.int32)
    table2 = table.reshape(1000000 // 2, 2 * D)
    emb = _sc_gather_pairs(table2, idx)
    return _tc_project(emb, W, b.reshape(1, D))


# speedup vs baseline: 1.7225x; 1.7225x over previous
import functools

import jax
import jax.numpy as jnp
from jax import lax
from jax.experimental import pallas as pl
from jax.experimental.pallas import tpu as pltpu
from jax.experimental.pallas import tpu_sc as plsc

D = 64
B = 16384


def _sc_gather(table, idx):
    info = plsc.get_sparse_core_info()
    nc, ns = info.num_cores, info.num_subcores
    nw = nc * ns
    b_per_w = B // nw

    mesh = plsc.VectorSubcoreMesh(core_axis_name="c", subcore_axis_name="s")

    @functools.partial(
        pl.kernel,
        mesh=mesh,
        out_type=jax.ShapeDtypeStruct((B, D), jnp.float32),
        scratch_types=[
            pltpu.VMEM((b_per_w,), jnp.int32),
            pltpu.VMEM((b_per_w, D), jnp.float32),
            pltpu.SemaphoreType.DMA,
        ],
    )
    def k(table_hbm, idx_hbm, out_hbm, idx_v, rows_v, sem):
        wid = lax.axis_index("s") * nc + lax.axis_index("c")
        base = wid * b_per_w
        pltpu.sync_copy(idx_hbm.at[pl.ds(base, b_per_w)], idx_v)

        def fire(g, _):
            vec = idx_v[pl.ds(g * 16, 16)]
            for lane in range(16):
                r = vec[lane]
                pltpu.make_async_copy(
                    table_hbm.at[r], rows_v.at[g * 16 + lane], sem
                ).start()
            return ()

        lax.fori_loop(0, b_per_w // 16, fire, ())
        pltpu.make_async_copy(
            table_hbm.at[pl.ds(0, b_per_w)], rows_v, sem
        ).wait()
        pltpu.sync_copy(rows_v, out_hbm.at[pl.ds(base, b_per_w)])

    return k(table, idx)


def _tc_project(emb, W, b2d):
    blk = 2048

    def body(emb_ref, w_ref, b_ref, out_ref):
        acc = lax.dot_general(
            emb_ref[...], w_ref[...],
            (((1,), (1,)), ((), ())),
            preferred_element_type=jnp.float32,
        )
        out_ref[...] = acc + b_ref[...]

    return pl.pallas_call(
        body,
        grid=(B // blk,),
        in_specs=[
            pl.BlockSpec((blk, D), lambda i: (i, 0)),
            pl.BlockSpec((D, D), lambda i: (0, 0)),
            pl.BlockSpec((1, D), lambda i: (0, 0)),
        ],
        out_specs=pl.BlockSpec((blk, D), lambda i: (i, 0)),
        out_shape=jax.ShapeDtypeStruct((B, D), jnp.float32),
    )(emb, W, b2d)


def kernel(l, table, W, b):
    idx = l.astype(jnp.int32)
    emb = _sc_gather(table, idx)
    return _tc_project(emb, W, b.reshape(1, D))
